# baseline (speedup 1.0000x reference)
import jax
import jax.numpy as jnp
from jax.experimental import pallas as pl
from jax.experimental.pallas import tpu as pltpu

_SDF = 128
_NPTS = 16
_BB = 16


def _raster_body(yt_ref, xt_ref, res_ref, org_ref, out_ref):
    for k in range(_BB):
        yt = yt_ref[k]
        xt = xt_ref[k]
        res = res_ref[k]
        org = org_ref[k]
        row = jnp.clip((yt / res[:, 0:1] + org[:, 0:1]).astype(jnp.int32), 0, _SDF - 1)
        col = jnp.clip((xt / res[:, 1:2] + org[:, 1:2]).astype(jnp.int32), 0, _SDF - 1)
        ci = jax.lax.broadcasted_iota(jnp.int32, (_NPTS, _SDF), 1)
        key = jnp.where(col == ci, row, -1)
        ri = jax.lax.broadcasted_iota(jnp.int32, (_SDF, _NPTS, _SDF), 0)
        out_ref[k] = (ri == key[None]).astype(jnp.float32)


def kernel(x, resolution, origin):
    b = x.shape[0]
    pts = x.reshape(b, _NPTS, 2)
    ys = jnp.broadcast_to(pts[:, :, 1][:, :, None], (b, _NPTS, _SDF))
    xs = jnp.broadcast_to(pts[:, :, 0][:, :, None], (b, _NPTS, _SDF))
    res3 = resolution.reshape(b, 1, 2)
    org3 = origin.reshape(b, 1, 2)
    out = pl.pallas_call(
        _raster_body,
        grid=(b // _BB,),
        in_specs=[
            pl.BlockSpec((_BB, _NPTS, _SDF), lambda i: (i, 0, 0)),
            pl.BlockSpec((_BB, _NPTS, _SDF), lambda i: (i, 0, 0)),
            pl.BlockSpec((_BB, 1, 2), lambda i: (i, 0, 0)),
            pl.BlockSpec((_BB, 1, 2), lambda i: (i, 0, 0)),
        ],
        out_specs=pl.BlockSpec((_BB, _SDF, _NPTS, _SDF), lambda i: (i, 0, 0, 0)),
        out_shape=jax.ShapeDtypeStruct((b, _SDF, _NPTS, _SDF), jnp.float32),
        compiler_params=pltpu.CompilerParams(
            dimension_semantics=("parallel",),
        ),
    )(ys, xs, res3, org3)
    return jnp.transpose(out, (0, 1, 3, 2))


# speedup vs baseline: 1.0095x; 1.0095x over previous
import jax
import jax.numpy as jnp
from jax.experimental import pallas as pl
from jax.experimental.pallas import tpu as pltpu

_SDF = 128
_NPTS = 16
_BB = 8


def _raster_body(yt_ref, xt_ref, res_ref, org_ref, out_ref):
    for k in range(_BB):
        yt = yt_ref[k]
        xt = xt_ref[k]
        res = res_ref[k]
        org = org_ref[k]
        row = jnp.clip((yt / res[:, 0:1] + org[:, 0:1]).astype(jnp.int32), 0, _SDF - 1)
        col = jnp.clip((xt / res[:, 1:2] + org[:, 1:2]).astype(jnp.int32), 0, _SDF - 1)
        ci = jax.lax.broadcasted_iota(jnp.int32, (_NPTS, _SDF), 1)
        key = jnp.where(col == ci, row, -1)
        ri = jax.lax.broadcasted_iota(jnp.int32, (_SDF, _NPTS, _SDF), 0)
        out_ref[k] = (ri == key[None]).astype(jnp.float32)


def kernel(x, resolution, origin):
    b = x.shape[0]
    pts = x.reshape(b, _NPTS, 2)
    ys = jnp.broadcast_to(pts[:, :, 1][:, :, None], (b, _NPTS, _SDF))
    xs = jnp.broadcast_to(pts[:, :, 0][:, :, None], (b, _NPTS, _SDF))
    res3 = resolution.reshape(b, 1, 2)
    org3 = origin.reshape(b, 1, 2)
    out = pl.pallas_call(
        _raster_body,
        grid=(b // _BB,),
        in_specs=[
            pl.BlockSpec((_BB, _NPTS, _SDF), lambda i: (i, 0, 0)),
            pl.BlockSpec((_BB, _NPTS, _SDF), lambda i: (i, 0, 0)),
            pl.BlockSpec((_BB, 1, 2), lambda i: (i, 0, 0)),
            pl.BlockSpec((_BB, 1, 2), lambda i: (i, 0, 0)),
        ],
        out_specs=pl.BlockSpec((_BB, _SDF, _NPTS, _SDF), lambda i: (i, 0, 0, 0)),
        out_shape=jax.ShapeDtypeStruct((b, _SDF, _NPTS, _SDF), jnp.float32),
        compiler_params=pltpu.CompilerParams(
            dimension_semantics=("parallel",),
        ),
    )(ys, xs, res3, org3)
    return jnp.transpose(out, (0, 1, 3, 2))


# speedup vs baseline: 1.0349x; 1.0252x over previous
import jax
import jax.numpy as jnp
from jax.experimental import pallas as pl
from jax.experimental.pallas import tpu as pltpu

_SDF = 128
_NPTS = 16
_BB = 8


def _raster_body(pts_ref, res_ref, org_ref, out_ref):
    for k in range(_BB):
        pk = pts_ref[k]
        res = res_ref[k]
        org = org_ref[k]
        row = jnp.clip((pk[:, 1:2] / res[:, 0:1] + org[:, 0:1]).astype(jnp.int32),
                       0, _SDF - 1)
        col = jnp.clip((pk[:, 0:1] / res[:, 1:2] + org[:, 1:2]).astype(jnp.int32),
                       0, _SDF - 1)
        ci = jax.lax.broadcasted_iota(jnp.int32, (_NPTS, _SDF), 1)
        key = jnp.where(col == ci, row, -1)
        ri = jax.lax.broadcasted_iota(jnp.int32, (_SDF, _NPTS, _SDF), 0)
        out_ref[k] = (ri == key[None]).astype(jnp.float32)


def kernel(x, resolution, origin):
    b = x.shape[0]
    pts = x.reshape(b, _NPTS, 2)
    res3 = resolution.reshape(b, 1, 2)
    org3 = origin.reshape(b, 1, 2)
    out = pl.pallas_call(
        _raster_body,
        grid=(b // _BB,),
        in_specs=[
            pl.BlockSpec((_BB, _NPTS, 2), lambda i: (i, 0, 0)),
            pl.BlockSpec((_BB, 1, 2), lambda i: (i, 0, 0)),
            pl.BlockSpec((_BB, 1, 2), lambda i: (i, 0, 0)),
        ],
        out_specs=pl.BlockSpec((_BB, _SDF, _NPTS, _SDF), lambda i: (i, 0, 0, 0)),
        out_shape=jax.ShapeDtypeStruct((b, _SDF, _NPTS, _SDF), jnp.float32),
        compiler_params=pltpu.CompilerParams(
            dimension_semantics=("parallel",),
        ),
    )(pts, res3, org3)
    return jnp.transpose(out, (0, 1, 3, 2))
